# initial kernel scaffold (unmeasured)
import jax
import jax.numpy as jnp
from jax import lax
from jax.experimental import pallas as pl
from jax.experimental.pallas import tpu as pltpu

N_DEV = 4
E_LOCAL = 4
E_TOTAL = 16


def kernel(x, router_W, route_idx, expert_W, shared_W):
    tokens, d = x.shape
    e_loc, _, h = expert_W.shape

    x16 = x.astype(jnp.bfloat16)
    rw16 = router_W.astype(jnp.bfloat16)
    ew16 = expert_W.astype(jnp.bfloat16)
    sw16 = shared_W.astype(jnp.bfloat16)

    def body(x_ref, rw_ref, idx_ref, ew_ref, sw_ref, out_ref,
             comm_ref, send_sems, recv_sems):
        my = lax.axis_index("i")
        right = lax.rem(my + 1, N_DEV)

        xv = x_ref[:]
        idx = idx_ref[:]

        s = jnp.dot(xv, rw_ref[:], preferred_element_type=jnp.float32)
        s = s - jnp.max(s, axis=-1, keepdims=True)
        p = jnp.exp(s)
        p = p / jnp.sum(p, axis=-1, keepdims=True)
        e_iota = lax.broadcasted_iota(jnp.int32, (1, E_TOTAL), 1)
        gate = jnp.sum(jnp.where(idx == e_iota, p, 0.0),
                       axis=-1, keepdims=True)

        acc = jnp.dot(xv, sw_ref[:], preferred_element_type=jnp.float32)

        def accum_group(acc, get_w, owner):
            for j in range(E_LOCAL):
                e = owner * E_LOCAL + j
                m = jnp.where(idx == e, gate, 0.0).astype(jnp.bfloat16)
                acc = acc + jnp.dot(xv * m, get_w(j),
                                    preferred_element_type=jnp.float32)
            return acc

        rdma0 = pltpu.make_async_remote_copy(
            src_ref=ew_ref, dst_ref=comm_ref.at[0],
            send_sem=send_sems.at[0], recv_sem=recv_sems.at[0],
            device_id=(right,), device_id_type=pl.DeviceIdType.MESH)
        rdma0.start()
        acc = accum_group(acc, lambda j: ew_ref[j], my)
        rdma0.wait()

        rdma1 = pltpu.make_async_remote_copy(
            src_ref=comm_ref.at[0], dst_ref=comm_ref.at[1],
            send_sem=send_sems.at[1], recv_sem=recv_sems.at[1],
            device_id=(right,), device_id_type=pl.DeviceIdType.MESH)
        rdma1.start()
        acc = accum_group(acc, lambda j: comm_ref[0, j], lax.rem(my + 3, N_DEV))
        rdma1.wait()

        rdma2 = pltpu.make_async_remote_copy(
            src_ref=comm_ref.at[1], dst_ref=comm_ref.at[0],
            send_sem=send_sems.at[2], recv_sem=recv_sems.at[2],
            device_id=(right,), device_id_type=pl.DeviceIdType.MESH)
        rdma2.start()
        acc = accum_group(acc, lambda j: comm_ref[1, j], lax.rem(my + 2, N_DEV))
        rdma2.wait()

        acc = accum_group(acc, lambda j: comm_ref[0, j], lax.rem(my + 1, N_DEV))

        out_ref[:] = acc

    return pl.pallas_call(
        body,
        out_shape=jax.ShapeDtypeStruct((tokens, h), jnp.float32),
        in_specs=[pl.BlockSpec(memory_space=pltpu.VMEM)] * 5,
        out_specs=pl.BlockSpec(memory_space=pltpu.VMEM),
        scratch_shapes=[
            pltpu.VMEM((2, e_loc, d, h), jnp.bfloat16),
            pltpu.SemaphoreType.DMA((3,)),
            pltpu.SemaphoreType.DMA((3,)),
        ],
    )(x16, rw16, route_idx, ew16, sw16)


# baseline (device time: 169986 ns/iter reference)
import jax
import jax.numpy as jnp
from jax import lax
from jax.experimental import pallas as pl
from jax.experimental.pallas import tpu as pltpu

N_DEV = 4
E_LOCAL = 4
E_TOTAL = 16


def kernel(x, router_W, route_idx, expert_W, shared_W):
    tokens, d = x.shape
    e_loc, _, h = expert_W.shape

    x16 = x.astype(jnp.bfloat16)
    rw16 = router_W.astype(jnp.bfloat16)
    ew16 = expert_W.astype(jnp.bfloat16)
    sw16 = shared_W.astype(jnp.bfloat16)

    def body(x_ref, rw_ref, idx_ref, ew_ref, sw_ref, out_ref,
             comm_ref, send_sems, recv_sems):
        my = lax.axis_index("i")
        right = lax.rem(my + 1, N_DEV)

        xv = x_ref[:]
        idx = idx_ref[:]

        s = jnp.dot(xv, rw_ref[:], preferred_element_type=jnp.float32)
        s = s - jnp.max(s, axis=-1, keepdims=True)
        p = jnp.exp(s)
        p = p / jnp.sum(p, axis=-1, keepdims=True)
        e_iota = lax.broadcasted_iota(jnp.int32, (1, E_TOTAL), 1)
        gate = jnp.sum(jnp.where(idx == e_iota, p, 0.0),
                       axis=-1, keepdims=True)

        out_ref[:] = jnp.dot(xv, sw_ref[:], preferred_element_type=jnp.float32)

        def accum_group(get_w, owner):
            for j in range(E_LOCAL):
                e = owner * E_LOCAL + j
                m = jnp.where(idx == e, gate, 0.0).astype(jnp.bfloat16)
                out_ref[:] = out_ref[:] + jnp.dot(
                    xv * m, get_w(j), preferred_element_type=jnp.float32)

        rdma0 = pltpu.make_async_remote_copy(
            src_ref=ew_ref, dst_ref=comm_ref.at[0],
            send_sem=send_sems.at[0], recv_sem=recv_sems.at[0],
            device_id=(right,), device_id_type=pl.DeviceIdType.MESH)
        rdma0.start()
        accum_group(lambda j: ew_ref[j], my)
        rdma0.wait()

        rdma1 = pltpu.make_async_remote_copy(
            src_ref=comm_ref.at[0], dst_ref=comm_ref.at[1],
            send_sem=send_sems.at[1], recv_sem=recv_sems.at[1],
            device_id=(right,), device_id_type=pl.DeviceIdType.MESH)
        rdma1.start()
        accum_group(lambda j: comm_ref[0, j], lax.rem(my + 3, N_DEV))
        rdma1.wait()

        rdma2 = pltpu.make_async_remote_copy(
            src_ref=comm_ref.at[1], dst_ref=comm_ref.at[0],
            send_sem=send_sems.at[2], recv_sem=recv_sems.at[2],
            device_id=(right,), device_id_type=pl.DeviceIdType.MESH)
        rdma2.start()
        accum_group(lambda j: comm_ref[1, j], lax.rem(my + 2, N_DEV))
        rdma2.wait()

        accum_group(lambda j: comm_ref[0, j], lax.rem(my + 1, N_DEV))

    return pl.pallas_call(
        body,
        out_shape=jax.ShapeDtypeStruct((tokens, h), jnp.float32),
        in_specs=[pl.BlockSpec(memory_space=pltpu.VMEM)] * 5,
        out_specs=pl.BlockSpec(memory_space=pltpu.VMEM),
        scratch_shapes=[
            pltpu.VMEM((2, e_loc, d, h), jnp.bfloat16),
            pltpu.SemaphoreType.DMA((3,)),
            pltpu.SemaphoreType.DMA((3,)),
        ],
    )(x16, rw16, route_idx, ew16, sw16)


# device time: 99166 ns/iter; 1.7142x vs baseline; 1.7142x over previous
import jax
import jax.numpy as jnp
from jax import lax
from jax.experimental import pallas as pl
from jax.experimental.pallas import tpu as pltpu

N_DEV = 4
E_LOCAL = 4
E_HALF = 2
E_TOTAL = 16


def kernel(x, router_W, route_idx, expert_W, shared_W):
    tokens, d = x.shape
    e_loc, _, h = expert_W.shape

    x16 = x.astype(jnp.bfloat16)
    rw16 = router_W.astype(jnp.bfloat16)
    ew16 = expert_W.astype(jnp.bfloat16)
    sw16 = shared_W.astype(jnp.bfloat16)

    def body(x_ref, rw_ref, idx_ref, ew_ref, sw_ref, out_ref,
             comm_r, comm_l, send_r, recv_r, send_l, recv_l):
        my = lax.axis_index("i")
        right = lax.rem(my + 1, N_DEV)
        left = lax.rem(my + 3, N_DEV)

        def hop(hn, src_r, dst_slot_r, src_l, dst_slot_l):
            r = pltpu.make_async_remote_copy(
                src_ref=src_r, dst_ref=comm_r.at[dst_slot_r],
                send_sem=send_r.at[hn], recv_sem=recv_r.at[hn],
                device_id=(right,), device_id_type=pl.DeviceIdType.MESH)
            l = pltpu.make_async_remote_copy(
                src_ref=src_l, dst_ref=comm_l.at[dst_slot_l],
                send_sem=send_l.at[hn], recv_sem=recv_l.at[hn],
                device_id=(left,), device_id_type=pl.DeviceIdType.MESH)
            r.start()
            l.start()
            return r, l

        r0, l0 = hop(0, ew_ref.at[pl.ds(0, E_HALF)], 0,
                     ew_ref.at[pl.ds(E_HALF, E_HALF)], 0)

        xv = x_ref[:]
        idx = idx_ref[:]

        s = jnp.dot(xv, rw_ref[:], preferred_element_type=jnp.float32)
        s = s - jnp.max(s, axis=-1, keepdims=True)
        p = jnp.exp(s)
        p = p / jnp.sum(p, axis=-1, keepdims=True)
        e_iota = lax.broadcasted_iota(jnp.int32, (1, E_TOTAL), 1)
        gate = jnp.sum(jnp.where(idx == e_iota, p, 0.0),
                       axis=-1, keepdims=True)

        out_ref[:] = jnp.dot(xv, sw_ref[:], preferred_element_type=jnp.float32)

        def accum(get_w, owner, j_base, n):
            for jj in range(n):
                e = owner * E_LOCAL + j_base + jj
                m = jnp.where(idx == e, gate, 0.0).astype(jnp.bfloat16)
                out_ref[:] = out_ref[:] + jnp.dot(
                    xv * m, get_w(jj), preferred_element_type=jnp.float32)

        accum(lambda j: ew_ref[j], my, 0, E_LOCAL)
        r0.wait()
        l0.wait()

        r1, l1 = hop(1, comm_r.at[0], 1, comm_l.at[0], 1)
        accum(lambda j: comm_r[0, j], lax.rem(my + 3, N_DEV), 0, E_HALF)
        accum(lambda j: comm_l[0, j], lax.rem(my + 1, N_DEV), E_HALF, E_HALF)
        r1.wait()
        l1.wait()

        r2, l2 = hop(2, comm_r.at[1], 0, comm_l.at[1], 0)
        accum(lambda j: comm_r[1, j], lax.rem(my + 2, N_DEV), 0, E_HALF)
        accum(lambda j: comm_l[1, j], lax.rem(my + 2, N_DEV), E_HALF, E_HALF)
        r2.wait()
        l2.wait()

        accum(lambda j: comm_r[0, j], lax.rem(my + 1, N_DEV), 0, E_HALF)
        accum(lambda j: comm_l[0, j], lax.rem(my + 3, N_DEV), E_HALF, E_HALF)

    return pl.pallas_call(
        body,
        out_shape=jax.ShapeDtypeStruct((tokens, h), jnp.float32),
        in_specs=[pl.BlockSpec(memory_space=pltpu.VMEM)] * 5,
        out_specs=pl.BlockSpec(memory_space=pltpu.VMEM),
        scratch_shapes=[
            pltpu.VMEM((2, E_HALF, d, h), jnp.bfloat16),
            pltpu.VMEM((2, E_HALF, d, h), jnp.bfloat16),
            pltpu.SemaphoreType.DMA((3,)),
            pltpu.SemaphoreType.DMA((3,)),
            pltpu.SemaphoreType.DMA((3,)),
            pltpu.SemaphoreType.DMA((3,)),
        ],
    )(x16, rw16, route_idx, ew16, sw16)
